# baseline (speedup 1.0000x reference)
import jax
import jax.numpy as jnp
from jax import lax
from jax.experimental import pallas as pl
from jax.experimental.pallas import tpu as pltpu
from jax.experimental.pallas import tpu_sc as plsc

NUM_CORES = 2
NUM_SUBCORES = 16
LANES = 16
NUM_WORKERS = NUM_CORES * NUM_SUBCORES

BATCH = 16384
FACTORS = 32
BPW = BATCH // NUM_WORKERS
CHUNK = 256
PASSES = BPW // CHUNK
CGROUPS = CHUNK // LANES
TPITCH = LANES + 1
NSEM = 8
ISTREAM = 128


def _mf_body(user_ref, item_ref, gu_hbm, gi_hbm, bu_hbm, bi_hbm,
             xui_out, gu_out, gi_out, bu_out, bi_out,
             idx_u, idx_i, rows_u, rows_i, bu_v, bi_v, xui_v, tbuf,
             *sems):
    sems_u = sems[:NSEM]
    sems_i = sems[NSEM:2 * NSEM]
    sem_b = sems[2 * NSEM]
    wid = lax.axis_index("s") * NUM_CORES + lax.axis_index("c")
    base = wid * BPW

    pltpu.sync_copy(user_ref.at[pl.ds(base, BPW)], idx_u)
    pltpu.sync_copy(item_ref.at[pl.ds(base, BPW)], idx_i)

    iota = lax.iota(jnp.int32, LANES)
    iota_t = iota * TPITCH

    def do_pass(p, carry):
        pbase = p * CHUNK

        def issue(g, c):
            rv_u = idx_u[pl.ds(pbase + g * LANES, LANES)]
            rv_i = idx_i[pl.ds(pbase + g * LANES, LANES)]
            for k in range(LANES):
                j = g * LANES + k
                s = k % NSEM
                pltpu.async_copy(gu_hbm.at[rv_u[k]], rows_u.at[j], sems_u[s])
                pltpu.async_copy(gi_hbm.at[rv_i[k]], rows_i.at[j], sems_i[s])
            return c

        lax.fori_loop(0, CGROUPS, issue, 0)

        bias_cps = []
        for h in range(CHUNK // ISTREAM):
            hsl = pl.ds(h * ISTREAM, ISTREAM)
            bias_cps.append(pltpu.async_copy(
                bu_hbm.at[idx_u.at[pl.ds(pbase + h * ISTREAM, ISTREAM)]],
                bu_v.at[hsl], sem_b))
            bias_cps.append(pltpu.async_copy(
                bi_hbm.at[idx_i.at[pl.ds(pbase + h * ISTREAM, ISTREAM)]],
                bi_v.at[hsl], sem_b))

        share = CHUNK // NSEM
        for s in range(NSEM):
            pltpu.make_async_copy(gu_out.at[pl.ds(0, share)],
                                  rows_u.at[pl.ds(0, share)], sems_u[s]).wait()
            pltpu.make_async_copy(gi_out.at[pl.ds(0, share)],
                                  rows_i.at[pl.ds(0, share)], sems_i[s]).wait()
        for cp in bias_cps:
            cp.wait()

        def group(g, c):
            row0 = g * LANES
            for r in range(LANES):
                row = row0 + r
                u0 = rows_u[row, pl.ds(0, LANES)]
                u1 = rows_u[row, pl.ds(LANES, LANES)]
                i0 = rows_i[row, pl.ds(0, LANES)]
                i1 = rows_i[row, pl.ds(LANES, LANES)]
                pp = u0 * i0 + u1 * i1
                plsc.store_scatter(tbuf, [iota_t + r], pp)
            acc = plsc.load_gather(tbuf, [iota])
            for k in range(1, LANES):
                acc = acc + plsc.load_gather(tbuf, [iota + (TPITCH * k)])
            sl = pl.ds(row0, LANES)
            xui_v[sl] = acc + bu_v[sl] + bi_v[sl]
            return c

        lax.fori_loop(0, CGROUPS, group, 0)

        out_sl = pl.ds(base + pbase, CHUNK)
        pltpu.sync_copy(rows_u, gu_out.at[out_sl])
        pltpu.sync_copy(rows_i, gi_out.at[out_sl])
        pltpu.sync_copy(bu_v, bu_out.at[out_sl])
        pltpu.sync_copy(bi_v, bi_out.at[out_sl])
        pltpu.sync_copy(xui_v, xui_out.at[out_sl])
        return carry

    lax.fori_loop(0, PASSES, do_pass, 0)


_mf_call = pl.kernel(
    _mf_body,
    mesh=plsc.VectorSubcoreMesh(core_axis_name="c", subcore_axis_name="s"),
    compiler_params=pltpu.CompilerParams(needs_layout_passes=False),
    out_type=(
        jax.ShapeDtypeStruct((BATCH,), jnp.float32),
        jax.ShapeDtypeStruct((BATCH, FACTORS), jnp.float32),
        jax.ShapeDtypeStruct((BATCH, FACTORS), jnp.float32),
        jax.ShapeDtypeStruct((BATCH,), jnp.float32),
        jax.ShapeDtypeStruct((BATCH,), jnp.float32),
    ),
    scratch_types=(
        pltpu.VMEM((BPW,), jnp.int32),
        pltpu.VMEM((BPW,), jnp.int32),
        pltpu.VMEM((CHUNK, FACTORS), jnp.float32),
        pltpu.VMEM((CHUNK, FACTORS), jnp.float32),
        pltpu.VMEM((CHUNK,), jnp.float32),
        pltpu.VMEM((CHUNK,), jnp.float32),
        pltpu.VMEM((CHUNK,), jnp.float32),
        pltpu.VMEM((LANES * TPITCH,), jnp.float32),
    ) + (pltpu.SemaphoreType.DMA,) * (2 * NSEM + 1),
)


@jax.jit
def kernel(user, item, Gu, Gi, Bu, Bi):
    return _mf_call(user, item, Gu, Gi, Bu, Bi)


# speedup vs baseline: 1.0973x; 1.0973x over previous
import jax
import jax.numpy as jnp
from jax import lax
from jax.experimental import pallas as pl
from jax.experimental.pallas import tpu as pltpu
from jax.experimental.pallas import tpu_sc as plsc

NUM_CORES = 2
NUM_SUBCORES = 16
LANES = 16
NUM_WORKERS = NUM_CORES * NUM_SUBCORES

BATCH = 16384
FACTORS = 32
BPW = BATCH // NUM_WORKERS
CHUNK = 256
PASSES = BPW // CHUNK
CGROUPS = CHUNK // LANES
TPITCH = LANES + 1
NSEM = 1
ISTREAM = 128


def _mf_body(user_ref, item_ref, gu_hbm, gi_hbm, bu_hbm, bi_hbm,
             xui_out, gu_out, gi_out, bu_out, bi_out,
             idx_u, idx_i, rows_u, rows_i, bu_v, bi_v, xui_v, tbuf,
             *sems):
    sems_u = sems[:NSEM]
    sems_i = sems[NSEM:2 * NSEM]
    sem_b = sems[2 * NSEM]
    wid = lax.axis_index("s") * NUM_CORES + lax.axis_index("c")
    base = wid * BPW

    pltpu.sync_copy(user_ref.at[pl.ds(base, BPW)], idx_u)
    pltpu.sync_copy(item_ref.at[pl.ds(base, BPW)], idx_i)

    iota = lax.iota(jnp.int32, LANES)
    iota_t = iota * TPITCH

    def do_pass(p, carry):
        pbase = p * CHUNK

        def issue(g, c):
            rv_u = idx_u[pl.ds(pbase + g * LANES, LANES)]
            rv_i = idx_i[pl.ds(pbase + g * LANES, LANES)]
            for k in range(LANES):
                j = g * LANES + k
                s = k % NSEM
                pltpu.async_copy(gu_hbm.at[rv_u[k]], rows_u.at[j], sems_u[s])
                pltpu.async_copy(gi_hbm.at[rv_i[k]], rows_i.at[j], sems_i[s])
            return c

        lax.fori_loop(0, CGROUPS, issue, 0)

        bias_cps = []
        for h in range(CHUNK // ISTREAM):
            hsl = pl.ds(h * ISTREAM, ISTREAM)
            bias_cps.append(pltpu.async_copy(
                bu_hbm.at[idx_u.at[pl.ds(pbase + h * ISTREAM, ISTREAM)]],
                bu_v.at[hsl], sem_b))
            bias_cps.append(pltpu.async_copy(
                bi_hbm.at[idx_i.at[pl.ds(pbase + h * ISTREAM, ISTREAM)]],
                bi_v.at[hsl], sem_b))

        share = CHUNK // NSEM
        for s in range(NSEM):
            pltpu.make_async_copy(gu_out.at[pl.ds(0, share)],
                                  rows_u.at[pl.ds(0, share)], sems_u[s]).wait()
            pltpu.make_async_copy(gi_out.at[pl.ds(0, share)],
                                  rows_i.at[pl.ds(0, share)], sems_i[s]).wait()
        for cp in bias_cps:
            cp.wait()

        def group(g, c):
            row0 = g * LANES
            for r in range(LANES):
                row = row0 + r
                u0 = rows_u[row, pl.ds(0, LANES)]
                u1 = rows_u[row, pl.ds(LANES, LANES)]
                i0 = rows_i[row, pl.ds(0, LANES)]
                i1 = rows_i[row, pl.ds(LANES, LANES)]
                pp = u0 * i0 + u1 * i1
                plsc.store_scatter(tbuf, [iota_t + r], pp)
            acc = plsc.load_gather(tbuf, [iota])
            for k in range(1, LANES):
                acc = acc + plsc.load_gather(tbuf, [iota + (TPITCH * k)])
            sl = pl.ds(row0, LANES)
            xui_v[sl] = acc + bu_v[sl] + bi_v[sl]
            return c

        lax.fori_loop(0, CGROUPS, group, 0)

        out_sl = pl.ds(base + pbase, CHUNK)
        pltpu.sync_copy(rows_u, gu_out.at[out_sl])
        pltpu.sync_copy(rows_i, gi_out.at[out_sl])
        pltpu.sync_copy(bu_v, bu_out.at[out_sl])
        pltpu.sync_copy(bi_v, bi_out.at[out_sl])
        pltpu.sync_copy(xui_v, xui_out.at[out_sl])
        return carry

    lax.fori_loop(0, PASSES, do_pass, 0)


_mf_call = pl.kernel(
    _mf_body,
    mesh=plsc.VectorSubcoreMesh(core_axis_name="c", subcore_axis_name="s"),
    compiler_params=pltpu.CompilerParams(needs_layout_passes=False),
    out_type=(
        jax.ShapeDtypeStruct((BATCH,), jnp.float32),
        jax.ShapeDtypeStruct((BATCH, FACTORS), jnp.float32),
        jax.ShapeDtypeStruct((BATCH, FACTORS), jnp.float32),
        jax.ShapeDtypeStruct((BATCH,), jnp.float32),
        jax.ShapeDtypeStruct((BATCH,), jnp.float32),
    ),
    scratch_types=(
        pltpu.VMEM((BPW,), jnp.int32),
        pltpu.VMEM((BPW,), jnp.int32),
        pltpu.VMEM((CHUNK, FACTORS), jnp.float32),
        pltpu.VMEM((CHUNK, FACTORS), jnp.float32),
        pltpu.VMEM((CHUNK,), jnp.float32),
        pltpu.VMEM((CHUNK,), jnp.float32),
        pltpu.VMEM((CHUNK,), jnp.float32),
        pltpu.VMEM((LANES * TPITCH,), jnp.float32),
    ) + (pltpu.SemaphoreType.DMA,) * (2 * NSEM + 1),
)


@jax.jit
def kernel(user, item, Gu, Gi, Bu, Bi):
    return _mf_call(user, item, Gu, Gi, Bu, Bi)


# speedup vs baseline: 1.1010x; 1.0034x over previous
import jax
import jax.numpy as jnp
from jax import lax
from jax.experimental import pallas as pl
from jax.experimental.pallas import tpu as pltpu
from jax.experimental.pallas import tpu_sc as plsc

NUM_CORES = 2
NUM_SUBCORES = 16
LANES = 16
NUM_WORKERS = NUM_CORES * NUM_SUBCORES

BATCH = 16384
FACTORS = 32
BPW = BATCH // NUM_WORKERS
CHUNK = 128
PASSES = BPW // CHUNK
CGROUPS = CHUNK // LANES
TPITCH = LANES + 1
ISTREAM = 128


def _mf_body(user_ref, item_ref, gu_hbm, gi_hbm, bu_hbm, bi_hbm,
             xui_out, gu_out, gi_out, bu_out, bi_out,
             idx_u, idx_i, rows_u, rows_i, bu_v, bi_v, xui_v, tbuf,
             sem_u0, sem_u1, sem_i0, sem_i1, sem_b):
    sems_u = (sem_u0, sem_u1)
    sems_i = (sem_i0, sem_i1)
    wid = lax.axis_index("s") * NUM_CORES + lax.axis_index("c")
    base = wid * BPW

    pltpu.sync_copy(user_ref.at[pl.ds(base, BPW)], idx_u)
    pltpu.sync_copy(item_ref.at[pl.ds(base, BPW)], idx_i)

    iota = lax.iota(jnp.int32, LANES)
    iota_t = iota * TPITCH

    bias_cps = []
    for h in range(BPW // ISTREAM):
        hsl = pl.ds(h * ISTREAM, ISTREAM)
        bias_cps.append(pltpu.async_copy(
            bu_hbm.at[idx_u.at[hsl]], bu_v.at[hsl], sem_b))
        bias_cps.append(pltpu.async_copy(
            bi_hbm.at[idx_i.at[hsl]], bi_v.at[hsl], sem_b))

    def issue(p, buf):
        def body(g, c):
            rv_u = idx_u[pl.ds(p * CHUNK + g * LANES, LANES)]
            rv_i = idx_i[pl.ds(p * CHUNK + g * LANES, LANES)]
            for k in range(LANES):
                j = g * LANES + k
                pltpu.async_copy(gu_hbm.at[rv_u[k]], rows_u.at[buf, j],
                                 sems_u[buf])
                pltpu.async_copy(gi_hbm.at[rv_i[k]], rows_i.at[buf, j],
                                 sems_i[buf])
            return c

        lax.fori_loop(0, CGROUPS, body, 0)

    issue(0, 0)

    for p in range(PASSES):
        buf = p % 2
        if p + 1 < PASSES:
            issue(p + 1, 1 - buf)

        pltpu.make_async_copy(gu_out.at[pl.ds(0, CHUNK)],
                              rows_u.at[buf], sems_u[buf]).wait()
        pltpu.make_async_copy(gi_out.at[pl.ds(0, CHUNK)],
                              rows_i.at[buf], sems_i[buf]).wait()
        if p == 0:
            for cp in bias_cps:
                cp.wait()

        def group(g, c):
            row0 = g * LANES
            for r in range(LANES):
                row = row0 + r
                u0 = rows_u[buf, row, pl.ds(0, LANES)]
                u1 = rows_u[buf, row, pl.ds(LANES, LANES)]
                i0 = rows_i[buf, row, pl.ds(0, LANES)]
                i1 = rows_i[buf, row, pl.ds(LANES, LANES)]
                pp = u0 * i0 + u1 * i1
                plsc.store_scatter(tbuf, [iota_t + r], pp)
            acc = plsc.load_gather(tbuf, [iota])
            for k in range(1, LANES):
                acc = acc + plsc.load_gather(tbuf, [iota + (TPITCH * k)])
            sl = pl.ds(p * CHUNK + row0, LANES)
            xui_v[sl] = acc + bu_v[sl] + bi_v[sl]
            return c

        lax.fori_loop(0, CGROUPS, group, 0)

        out_sl = pl.ds(base + p * CHUNK, CHUNK)
        pltpu.sync_copy(rows_u.at[buf], gu_out.at[out_sl])
        pltpu.sync_copy(rows_i.at[buf], gi_out.at[out_sl])

    out_sl = pl.ds(base, BPW)
    pltpu.sync_copy(bu_v, bu_out.at[out_sl])
    pltpu.sync_copy(bi_v, bi_out.at[out_sl])
    pltpu.sync_copy(xui_v, xui_out.at[out_sl])


_mf_call = pl.kernel(
    _mf_body,
    mesh=plsc.VectorSubcoreMesh(core_axis_name="c", subcore_axis_name="s"),
    compiler_params=pltpu.CompilerParams(needs_layout_passes=False),
    out_type=(
        jax.ShapeDtypeStruct((BATCH,), jnp.float32),
        jax.ShapeDtypeStruct((BATCH, FACTORS), jnp.float32),
        jax.ShapeDtypeStruct((BATCH, FACTORS), jnp.float32),
        jax.ShapeDtypeStruct((BATCH,), jnp.float32),
        jax.ShapeDtypeStruct((BATCH,), jnp.float32),
    ),
    scratch_types=(
        pltpu.VMEM((BPW,), jnp.int32),
        pltpu.VMEM((BPW,), jnp.int32),
        pltpu.VMEM((2, CHUNK, FACTORS), jnp.float32),
        pltpu.VMEM((2, CHUNK, FACTORS), jnp.float32),
        pltpu.VMEM((BPW,), jnp.float32),
        pltpu.VMEM((BPW,), jnp.float32),
        pltpu.VMEM((BPW,), jnp.float32),
        pltpu.VMEM((LANES * TPITCH,), jnp.float32),
        pltpu.SemaphoreType.DMA,
        pltpu.SemaphoreType.DMA,
        pltpu.SemaphoreType.DMA,
        pltpu.SemaphoreType.DMA,
        pltpu.SemaphoreType.DMA,
    ),
)


@jax.jit
def kernel(user, item, Gu, Gi, Bu, Bi):
    return _mf_call(user, item, Gu, Gi, Bu, Bi)
